# baseline (speedup 1.0000x reference)
import functools

import jax
import jax.numpy as jnp
from jax import lax
from jax.experimental import pallas as pl
from jax.experimental.pallas import tpu as pltpu
from jax.experimental.pallas import tpu_sc as plsc

LANES = 16


@functools.lru_cache(maxsize=None)
def _make_sc_embed(R, NW, NC, S, K, D, NBUF, BT):
    mesh = plsc.VectorSubcoreMesh(core_axis_name="c", subcore_axis_name="s")
    NVEC = D // LANES
    CR = S * K
    CHUNKS = R // (NW * CR)
    LA = 96
    LB = CR - LA

    scratch = [
        pltpu.VMEM((S * K, BT), jnp.int32),
        pltpu.VMEM((CHUNKS, LA), jnp.int32),
        pltpu.VMEM((CHUNKS, LB), jnp.int32),
        pltpu.VMEM((56, D), jnp.float32),
    ]
    scratch += [pltpu.VMEM((CR, D), jnp.float32) for _ in range(NBUF)]
    scratch += [pltpu.SemaphoreType.DMA for _ in range(2 * NBUF)]

    @functools.partial(
        pl.kernel,
        mesh=mesh,
        out_type=jax.ShapeDtypeStruct((R, D), jnp.float32),
        scratch_types=scratch,
        compiler_params=pltpu.CompilerParams(needs_layout_passes=False),
    )
    def sc_embed(raw_hbm, table_hbm, pe_hbm, out_hbm,
                 raw_v, idxa_v, idxb_v, pe_v, *rest):
        bufs = rest[:NBUF]
        gsems = rest[NBUF:2 * NBUF]
        wsems = rest[2 * NBUF:3 * NBUF]

        wid = lax.axis_index("s") * NC + lax.axis_index("c")

        bt = lax.div(wid, 4)
        for s_i in range(S):
            pltpu.async_copy(raw_hbm.at[s_i, bt],
                             raw_v.at[pl.ds(s_i * K, K)], gsems[0])
        for s_i in range(S):
            pltpu.make_async_copy(raw_hbm.at[s_i, bt],
                                  raw_v.at[pl.ds(s_i * K, K)],
                                  gsems[0]).wait()
        pltpu.sync_copy(pe_hbm.at[pl.ds(0, 56)], pe_v)

        bl0 = lax.rem(wid, 4) * 32
        lane = lax.broadcasted_iota(jnp.int32, (LANES,), 0)

        def reorder_into(dst2d, width, jg_off):
            nfull = width // LANES
            tail = width - nfull * LANES

            def body(c, carry):
                bv = lane * 0 + (c + bl0)
                for t in range(nfull):
                    jg = lane + (t * LANES + jg_off)
                    vals = plsc.load_gather(raw_v, [jg, bv])
                    dst2d[c, pl.ds(t * LANES, LANES)] = vals
                if tail:
                    j = lane + nfull * LANES
                    msk = j < width
                    jg = lax.min(j + jg_off, CR - 1)
                    vals = plsc.load_gather(raw_v, [jg, bv], mask=msk)
                    cv = lane * 0 + c
                    plsc.store_scatter(dst2d, [cv, j], vals, mask=msk)
                return carry

            lax.fori_loop(0, CHUNKS, body, 0, unroll=False)

        reorder_into(idxa_v, LA, 0)
        reorder_into(idxb_v, LB, LA)

        def gather_start(g, b):
            pltpu.async_copy(table_hbm.at[idxa_v.at[g]],
                             bufs[b].at[pl.ds(0, LA)], gsems[b])
            pltpu.async_copy(table_hbm.at[idxb_v.at[g]],
                             bufs[b].at[pl.ds(LA, LB)], gsems[b])

        def gather_wait(g, b):
            pltpu.make_async_copy(
                table_hbm.at[idxa_v.at[g]],
                bufs[b].at[pl.ds(0, LA)], gsems[b]).wait()
            pltpu.make_async_copy(
                table_hbm.at[idxb_v.at[g]],
                bufs[b].at[pl.ds(LA, LB)], gsems[b]).wait()

        def out_window(g):
            return out_hbm.at[pl.ds((wid * CHUNKS + g) * CR, CR)]

        def write_start(g, b):
            pltpu.async_copy(bufs[b], out_window(g), wsems[b])

        def write_wait(g, b):
            pltpu.make_async_copy(bufs[b], out_window(g), wsems[b]).wait()

        def add_pe(b):
            buf = bufs[b]

            def body(i, carry):
                pes = [pe_v[i, pl.ds(v * LANES, LANES)] for v in range(NVEC)]
                for j in range(K):
                    r = i * K + j
                    for v in range(NVEC):
                        plsc.addupdate(
                            buf.at[r, pl.ds(v * LANES, LANES)], pes[v])
                return carry

            lax.fori_loop(0, S, body, 0, unroll=False)

        for b in range(NBUF):
            gather_start(b, b)

        for g in range(CHUNKS):
            if 1 <= g <= CHUNKS - NBUF:
                bp = (g - 1) % NBUF
                write_wait(g - 1, bp)
                gather_start(g - 1 + NBUF, bp)
            b = g % NBUF
            gather_wait(g, b)
            add_pe(b)
            write_start(g, b)

        for h in range(CHUNKS - NBUF, CHUNKS):
            write_wait(h, h % NBUF)

    return sc_embed


def kernel(inputs, token_table, pos_embedding):
    B, K, S = inputs.shape
    V, D = token_table.shape

    info = plsc.get_sparse_core_info()
    NW = info.num_cores * info.num_subcores
    NC = info.num_cores

    R = B * K * S
    BT = 128
    NBUF = 3

    raw = inputs.reshape(B // BT, BT, K, S).transpose(3, 0, 2, 1)

    fn = _make_sc_embed(R, NW, NC, S, K, D, NBUF, BT)
    out = fn(raw, token_table, pos_embedding)
    return out.reshape(B, S, K, D).transpose(0, 2, 1, 3)


# speedup vs baseline: 1.0187x; 1.0187x over previous
import functools

import jax
import jax.numpy as jnp
from jax import lax
from jax.experimental import pallas as pl
from jax.experimental.pallas import tpu as pltpu
from jax.experimental.pallas import tpu_sc as plsc

LANES = 16


@functools.lru_cache(maxsize=None)
def _make_sc_embed(R, NW, NC, S, K, D, NBUF, BT):
    mesh = plsc.VectorSubcoreMesh(core_axis_name="c", subcore_axis_name="s")
    NVEC = D // LANES
    CR = S * K
    CHUNKS = R // (NW * CR)
    LA = 96
    LB = CR - LA

    scratch = [
        pltpu.VMEM((S * K, BT), jnp.int32),
        pltpu.VMEM((CHUNKS, LA), jnp.int32),
        pltpu.VMEM((CHUNKS, LB), jnp.int32),
        pltpu.VMEM((S, D), jnp.float32),
    ]
    scratch += [pltpu.VMEM((CR, D), jnp.float32) for _ in range(NBUF)]
    scratch += [pltpu.SemaphoreType.DMA for _ in range(2 * NBUF)]

    @functools.partial(
        pl.kernel,
        mesh=mesh,
        out_type=jax.ShapeDtypeStruct((R, D), jnp.float32),
        scratch_types=scratch,
        compiler_params=pltpu.CompilerParams(needs_layout_passes=False),
    )
    def sc_embed(raw_hbm, table_hbm, pe_hbm, out_hbm,
                 raw_v, idxa_v, idxb_v, pe_v, *rest):
        bufs = rest[:NBUF]
        gsems = rest[NBUF:2 * NBUF]
        wsems = rest[2 * NBUF:3 * NBUF]

        wid = lax.axis_index("s") * NC + lax.axis_index("c")

        bt = lax.div(wid, 4)
        for s_i in range(S):
            pltpu.async_copy(raw_hbm.at[s_i, bt],
                             raw_v.at[pl.ds(s_i * K, K)], gsems[0])
        for s_i in range(S):
            pltpu.make_async_copy(raw_hbm.at[s_i, bt],
                                  raw_v.at[pl.ds(s_i * K, K)],
                                  gsems[0]).wait()
        pe_cp = pltpu.async_copy(pe_hbm, pe_v, wsems[0])

        bl0 = lax.rem(wid, 4) * 32
        lane = lax.broadcasted_iota(jnp.int32, (LANES,), 0)

        def reorder_into(dst2d, width, jg_off):
            nfull = width // LANES
            tail = width - nfull * LANES

            def body(c, carry):
                bv = lane * 0 + (c + bl0)
                for t in range(nfull):
                    jg = lane + (t * LANES + jg_off)
                    vals = plsc.load_gather(raw_v, [jg, bv])
                    dst2d[c, pl.ds(t * LANES, LANES)] = vals
                if tail:
                    j = lane + nfull * LANES
                    msk = j < width
                    jg = lax.min(j + jg_off, CR - 1)
                    vals = plsc.load_gather(raw_v, [jg, bv], mask=msk)
                    cv = lane * 0 + c
                    plsc.store_scatter(dst2d, [cv, j], vals, mask=msk)
                return carry

            lax.fori_loop(0, CHUNKS, body, 0, unroll=False)

        reorder_into(idxa_v, LA, 0)
        reorder_into(idxb_v, LB, LA)

        def gather_start(g, b):
            pltpu.async_copy(table_hbm.at[idxa_v.at[g]],
                             bufs[b].at[pl.ds(0, LA)], gsems[b])
            pltpu.async_copy(table_hbm.at[idxb_v.at[g]],
                             bufs[b].at[pl.ds(LA, LB)], gsems[b])

        def gather_wait(g, b):
            pltpu.make_async_copy(
                table_hbm.at[idxa_v.at[g]],
                bufs[b].at[pl.ds(0, LA)], gsems[b]).wait()
            pltpu.make_async_copy(
                table_hbm.at[idxb_v.at[g]],
                bufs[b].at[pl.ds(LA, LB)], gsems[b]).wait()

        def out_window(g):
            return out_hbm.at[pl.ds((wid * CHUNKS + g) * CR, CR)]

        def write_start(g, b):
            pltpu.async_copy(bufs[b], out_window(g), wsems[b])

        def write_wait(g, b):
            pltpu.make_async_copy(bufs[b], out_window(g), wsems[b]).wait()

        def add_pe(b):
            buf = bufs[b]

            def body(i, carry):
                pes = [pe_v[i, pl.ds(v * LANES, LANES)] for v in range(NVEC)]
                for j in range(K):
                    r = i * K + j
                    for v in range(NVEC):
                        plsc.addupdate(
                            buf.at[r, pl.ds(v * LANES, LANES)], pes[v])
                return carry

            lax.fori_loop(0, S, body, 0, unroll=False)

        for b in range(NBUF):
            gather_start(b, b)
        pe_cp.wait()

        for g in range(CHUNKS):
            if 1 <= g <= CHUNKS - NBUF:
                bp = (g - 1) % NBUF
                write_wait(g - 1, bp)
                gather_start(g - 1 + NBUF, bp)
            b = g % NBUF
            gather_wait(g, b)
            add_pe(b)
            write_start(g, b)

        for h in range(CHUNKS - NBUF, CHUNKS):
            write_wait(h, h % NBUF)

    return sc_embed


def kernel(inputs, token_table, pos_embedding):
    B, K, S = inputs.shape
    V, D = token_table.shape

    info = plsc.get_sparse_core_info()
    NW = info.num_cores * info.num_subcores
    NC = info.num_cores

    R = B * K * S
    BT = 128
    NBUF = 3

    raw = inputs.reshape(B // BT, BT, K, S).transpose(3, 0, 2, 1)
    pe_seq = pos_embedding[:S]

    fn = _make_sc_embed(R, NW, NC, S, K, D, NBUF, BT)
    out = fn(raw, token_table, pe_seq)
    return out.reshape(B, S, K, D).transpose(0, 2, 1, 3)
